# baseline (speedup 1.0000x reference)
import functools

import jax
import jax.numpy as jnp
from jax import lax
from jax.experimental import pallas as pl
from jax.experimental.pallas import tpu as pltpu
from jax.experimental.pallas import tpu_sc as plsc

N = 50000
F_IN = 7
H = 64
C = 2
G = 512
E = 800000

NC = 2
NS = 16
CH = 128
EP = 851968
NCHUNK = EP // CH
RPW = NCHUNK // (NC * NS)
BLK = 16
HALF = 25600
NP = 2 * HALF
ACC = 26624
RCAP = 27648
RBLK = 8
NREG = NC * NS
BP = 65536
GP = 640
DSTRIPE = HALF // NS

_mesh = plsc.VectorSubcoreMesh(core_axis_name="c", subcore_axis_name="s")
_f32 = jnp.float32
_SC_PARAMS = pltpu.CompilerParams(needs_layout_passes=False,
                                  use_tc_tiling_on_sc=False)


@functools.partial(
    pl.kernel,
    out_type=(jax.ShapeDtypeStruct((NC * NREG * RCAP,), jnp.int32),
              jax.ShapeDtypeStruct((NC * NREG * RCAP,), jnp.int32),
              jax.ShapeDtypeStruct((NREG * 8,), jnp.int32)),
    mesh=_mesh,
    compiler_params=_SC_PARAMS,
    scratch_types=[
        pltpu.VMEM((RCAP,), jnp.int32),
        pltpu.VMEM((RCAP,), jnp.int32),
        pltpu.VMEM((RCAP,), jnp.int32),
        pltpu.VMEM((RCAP,), jnp.int32),
        pltpu.VMEM((BLK, CH), jnp.int32),
        pltpu.VMEM((BLK, CH), jnp.int32),
        pltpu.VMEM((16,), jnp.int32),
    ],
)
def _partition(src_hbm, dst_hbm, srcp_hbm, dstp_hbm, ecnt_hbm,
               b0s, b0d, b1s, b1d, sblk, dblk, cbuf):
    c = lax.axis_index("c")
    s = lax.axis_index("s")
    w = s * NC + c
    iota16 = lax.broadcasted_iota(jnp.int32, (16,), 0)

    def _blk(b, offs):
        off0, off1 = offs
        row0 = w * RPW + b * BLK
        pltpu.sync_copy(src_hbm.at[pl.ds(row0, BLK)], sblk)
        pltpu.sync_copy(dst_hbm.at[pl.ds(row0, BLK)], dblk)
        for r in range(BLK):
            for k in range(CH // 16):
                sv = sblk[r, pl.ds(k * 16, 16)]
                dv = dblk[r, pl.ds(k * 16, 16)]
                m0 = dv < HALF
                m1 = jnp.logical_not(m0)
                pc0 = plsc.all_reduce_population_count(m0)[0]
                plsc.store_compressed(b0s.at[pl.ds(off0, 16)], sv, mask=m0)
                plsc.store_compressed(b0d.at[pl.ds(off0, 16)], dv, mask=m0)
                plsc.store_compressed(b1s.at[pl.ds(off1, 16)], sv, mask=m1)
                plsc.store_compressed(b1d.at[pl.ds(off1, 16)], dv - HALF, mask=m1)
                off0 = off0 + pc0
                off1 = off1 + (16 - pc0)
        return (off0, off1)

    zero = jnp.zeros((), jnp.int32)
    off0, off1 = lax.fori_loop(0, RPW // BLK, _blk, (zero, zero))

    dum_s = jnp.zeros((16,), jnp.int32)
    dum_d = jnp.full((16,), HALF, jnp.int32)
    for buf, off, dum in ((b0s, off0, dum_s), (b0d, off0, dum_d),
                          (b1s, off1, dum_s), (b1d, off1, dum_d)):
        a0 = (off // 16) * 16
        v = buf[pl.ds(a0, 16)]
        buf[pl.ds(a0, 16)] = jnp.where(iota16 < (off - a0), v, dum)
        for i in range(1, 64):
            buf[pl.ds(a0 + i * 16, 16)] = dum

    base0 = (0 * NREG + w) * RCAP
    base1 = (1 * NREG + w) * RCAP
    pltpu.sync_copy(b0s, srcp_hbm.at[pl.ds(base0, RCAP)])
    pltpu.sync_copy(b1s, srcp_hbm.at[pl.ds(base1, RCAP)])
    pltpu.sync_copy(b0d, dstp_hbm.at[pl.ds(base0, RCAP)])
    pltpu.sync_copy(b1d, dstp_hbm.at[pl.ds(base1, RCAP)])
    cv = jnp.where(iota16 == 0, off0, jnp.where(iota16 == 1, off1, 0))
    cbuf[...] = cv
    pltpu.sync_copy(cbuf.at[pl.ds(0, 8)], ecnt_hbm.at[pl.ds(w * 8, 8)])


@functools.partial(
    pl.kernel,
    out_type=(jax.ShapeDtypeStruct((NC * NS * N,), _f32),
              jax.ShapeDtypeStruct((NC * NS * G,), _f32)),
    mesh=_mesh,
    compiler_params=_SC_PARAMS,
    scratch_types=[
        pltpu.VMEM((N,), _f32),
        pltpu.VMEM((G,), _f32),
        pltpu.VMEM((BLK, CH), jnp.int32),
    ],
)
def _deg_counts(dst_hbm, batch_hbm, degp_hbm, cntp_hbm, deg_v, cnt_v, blk_v):
    c = lax.axis_index("c")
    s = lax.axis_index("s")
    w = s * NC + c
    zero16 = jnp.zeros((16,), _f32)
    one16 = jnp.ones((16,), _f32)

    def _z(i, _):
        deg_v[pl.ds(i * 16, 16)] = zero16
        return 0

    lax.fori_loop(0, N // 16, _z, 0)
    for i in range(G // 16):
        cnt_v[pl.ds(i * 16, 16)] = zero16

    def _deg_blk(b, _):
        pltpu.sync_copy(dst_hbm.at[pl.ds(w * RPW + b * BLK, BLK)], blk_v)
        for r in range(BLK):
            for k in range(CH // 16):
                v = blk_v[r, pl.ds(k * 16, 16)]
                plsc.addupdate_scatter(deg_v, [v], one16, mask=v < N)
        return 0

    lax.fori_loop(0, RPW // BLK, _deg_blk, 0)

    nrows = (BP // CH) // (NC * NS)
    pltpu.sync_copy(batch_hbm.at[pl.ds(w * nrows, nrows)], blk_v)
    for r in range(nrows):
        for k in range(CH // 16):
            v = blk_v[r, pl.ds(k * 16, 16)]
            plsc.addupdate_scatter(cnt_v, [v], one16, mask=v < G)

    def _wr(i, _):
        pltpu.sync_copy(deg_v.at[pl.ds(i * 2000, 2000)],
                        degp_hbm.at[pl.ds(i * (NC * NS * 2000) + w * 2000,
                                          2000)])
        return 0

    lax.fori_loop(0, N // 2000, _wr, 0)
    pltpu.sync_copy(cnt_v, cntp_hbm.at[pl.ds(w * G, G)])


def _agg_half_body(srcp_hbm, dstp_hbm, ecnt_v, q_hbm, zeros_hbm, acc_sh,
                   sb8, db8, rb, gsems, ssems):
    c = lax.axis_index("c")
    s = lax.axis_index("s")

    def _zc(i, _):
        pltpu.sync_copy(zeros_hbm, acc_sh.at[pl.ds(s * (ACC // NS) + i * CH,
                                                   CH)])
        return 0

    lax.fori_loop(0, (ACC // NS) // CH, _zc, 0)
    plsc.subcore_barrier()

    for i in range(2):
        r = s * 2 + i
        cnt = plsc.load_gather(ecnt_v, [lax.broadcast(r * 8 + c, (16,))])[0]
        nblk = (cnt + (RBLK * CH - 1)) // (RBLK * CH)
        row0 = (c * NREG + r) * (RCAP // CH)

        def _blk(b, _):
            pltpu.sync_copy(srcp_hbm.at[pl.ds(row0 + b * RBLK, RBLK)], sb8)
            pltpu.sync_copy(dstp_hbm.at[pl.ds(row0 + b * RBLK, RBLK)], db8)
            gd = [None, None]
            gd[0] = pltpu.async_copy(q_hbm.at[sb8.at[0]], rb.at[0], gsems[0])
            for k in range(RBLK):
                if k + 1 < RBLK:
                    q = (k + 1) % 2
                    gd[q] = pltpu.async_copy(q_hbm.at[sb8.at[k + 1]],
                                             rb.at[q], gsems[q])
                gd[k % 2].wait()
                pltpu.sync_copy(rb.at[k % 2], acc_sh.at[db8.at[k]], add=True)
            return 0

        lax.fori_loop(0, nblk, _blk, 0)


def _agg_scratch(width):
    return [
        pltpu.VMEM_SHARED((ACC, width), _f32),
        pltpu.VMEM((RBLK, CH), jnp.int32),
        pltpu.VMEM((RBLK, CH), jnp.int32),
        pltpu.VMEM((2, CH, width), _f32),
        pltpu.VMEM((NREG * 8,), jnp.int32),
        pltpu.SemaphoreType.DMA,
        pltpu.SemaphoreType.DMA,
        pltpu.SemaphoreType.DMA,
        pltpu.SemaphoreType.DMA,
    ]


def _make_agg(width):
    @functools.partial(
        pl.kernel,
        out_type=jax.ShapeDtypeStruct((NP, width), _f32),
        mesh=_mesh,
        compiler_params=_SC_PARAMS,
        scratch_types=_agg_scratch(width),
    )
    def _agg(zeros_hbm, srcp_hbm, dstp_hbm, ecnt_hbm, q_hbm, out_hbm,
             acc_sh, sb8, db8, rb, ecnt_v, gsem0, gsem1, ssem0, ssem1):
        c = lax.axis_index("c")
        s = lax.axis_index("s")
        pltpu.sync_copy(ecnt_hbm, ecnt_v)
        _agg_half_body(srcp_hbm, dstp_hbm, ecnt_v, q_hbm, zeros_hbm, acc_sh,
                       sb8, db8, rb, (gsem0, gsem1), (ssem0, ssem1))
        plsc.subcore_barrier()
        pltpu.sync_copy(acc_sh.at[pl.ds(s * DSTRIPE, DSTRIPE)],
                        out_hbm.at[pl.ds(c * HALF + s * DSTRIPE, DSTRIPE)])

    return _agg


_agg8 = _make_agg(8)
_agg64 = _make_agg(64)


@functools.partial(
    pl.kernel,
    out_type=jax.ShapeDtypeStruct((NC * G, H), _f32),
    mesh=_mesh,
    compiler_params=_SC_PARAMS,
    scratch_types=_agg_scratch(H) + [
        pltpu.VMEM_SHARED((GP, H), _f32),
        pltpu.VMEM((CH,), _f32),
        pltpu.VMEM((CH,), jnp.int32),
        pltpu.VMEM((64,), jnp.int32),
    ],
)
def _agg64_pool(zeros_hbm, srcp_hbm, dstp_hbm, ecnt_hbm, q_hbm, dis_hbm,
                batch_hbm, out_hbm, acc_sh, sb8, db8, rb, ecnt_v,
                gsem0, gsem1, ssem0, ssem1, pool_sh, dbuf, bidx, bidx64):
    c = lax.axis_index("c")
    s = lax.axis_index("s")
    gstripe = GP // NS
    pltpu.sync_copy(zeros_hbm.at[pl.ds(0, gstripe)],
                    pool_sh.at[pl.ds(s * gstripe, gstripe)])
    pltpu.sync_copy(ecnt_hbm, ecnt_v)
    _agg_half_body(srcp_hbm, dstp_hbm, ecnt_v, q_hbm, zeros_hbm, acc_sh,
                   sb8, db8, rb, (gsem0, gsem1), (ssem0, ssem1))
    plsc.subcore_barrier()

    pz = rb.at[0]

    def _chunk(t, nr, idx_ref):
        local0 = s * DSTRIPE + t * CH
        node0 = c * HALF + local0
        pltpu.sync_copy(acc_sh.at[pl.ds(local0, nr)], pz.at[pl.ds(0, nr)])
        pltpu.sync_copy(dis_hbm.at[pl.ds(node0, nr)], dbuf.at[pl.ds(0, nr)])
        pltpu.sync_copy(batch_hbm.at[pl.ds(node0, nr)], idx_ref)

        def _srow(j, _):
            idx16 = lax.broadcast(j, (16,))
            dsp = plsc.load_gather(dbuf, [idx16])
            for f in range(H // 16):
                pz[j, pl.ds(f * 16, 16)] = pz[j, pl.ds(f * 16, 16)] * dsp
            return 0

        lax.fori_loop(0, nr, _srow, 0)
        pltpu.sync_copy(pz.at[pl.ds(0, nr)], pool_sh.at[idx_ref], add=True)

    def _floop(t, _):
        _chunk(t, CH, bidx)
        return 0

    lax.fori_loop(0, DSTRIPE // CH, _floop, 0)
    _chunk(DSTRIPE // CH, DSTRIPE - (DSTRIPE // CH) * CH, bidx64)
    plsc.subcore_barrier()
    gd = G // NS
    pltpu.sync_copy(pool_sh.at[pl.ds(s * gd, gd)],
                    out_hbm.at[pl.ds(c * G + s * gd, gd)])


_NB = 2000
_NBLK = N // _NB


def _tc0_body(degp_ref, x_ref, dis_ref, t0_ref):
    deg = jnp.sum(degp_ref[0], axis=0)
    dis = lax.rsqrt(deg)
    dis_ref[...] = dis[:, None]
    t0 = x_ref[...] * dis[:, None]
    t0_ref[...] = jnp.concatenate([t0, jnp.zeros((_NB, 1), _f32)], axis=1)


def _tc0(degp, x):
    return pl.pallas_call(
        _tc0_body,
        grid=(_NBLK,),
        in_specs=[
            pl.BlockSpec((1, NC * NS, _NB), lambda i: (i, 0, 0)),
            pl.BlockSpec((_NB, F_IN), lambda i: (i, 0)),
        ],
        out_specs=[
            pl.BlockSpec((_NB, 1), lambda i: (i, 0)),
            pl.BlockSpec((_NB, 8), lambda i: (i, 0)),
        ],
        out_shape=[
            jax.ShapeDtypeStruct((N, 1), _f32),
            jax.ShapeDtypeStruct((N, 8), _f32),
        ],
    )(degp, x)


def _tc_layer_body(ap_ref, dis_ref, w_ref, b_ref, q_ref):
    a = ap_ref[...]
    dis = dis_ref[...]
    z = a * dis
    h = jnp.dot(z, w_ref[...], preferred_element_type=_f32)
    h = jnp.maximum(h + b_ref[...], 0.0)
    q_ref[...] = h * dis


def _tc_layer(ap, dis, w, b):
    kin = ap.shape[-1]
    return pl.pallas_call(
        _tc_layer_body,
        grid=(_NBLK,),
        in_specs=[
            pl.BlockSpec((_NB, kin), lambda i: (i, 0)),
            pl.BlockSpec((_NB, 1), lambda i: (i, 0)),
            pl.BlockSpec(w.shape, lambda i: (0, 0)),
            pl.BlockSpec((1, H), lambda i: (0, 0)),
        ],
        out_specs=pl.BlockSpec((_NB, H), lambda i: (i, 0)),
        out_shape=jax.ShapeDtypeStruct((N, H), _f32),
    )(ap, dis, w, b)


def _tc_final_body(p_ref, cntp_ref, w3_ref, b3_ref, wl_ref, bl_ref, out_ref):
    counts = jnp.sum(cntp_ref[...], axis=0)
    p64 = p_ref[0] + p_ref[1]
    hs = jnp.dot(p64, w3_ref[...], preferred_element_type=_f32)
    hs = hs + counts[:, None] * b3_ref[...]
    pooled = hs / jnp.maximum(counts, 1.0)[:, None]
    out = jnp.dot(pooled, wl_ref[...], preferred_element_type=_f32)
    out_ref[...] = out + bl_ref[...]


def _tc_final(p, cntp, w3, b3, wl, bl):
    return pl.pallas_call(
        _tc_final_body,
        out_shape=jax.ShapeDtypeStruct((G, C), _f32),
    )(p, cntp, w3, b3, wl, bl)


def kernel(x, edge_index, batch, W1, b1, W2, b2, W3, b3, Wlin, blin):
    loop = jnp.arange(N, dtype=jnp.int32)
    pad = EP - (E + N)
    src = jnp.concatenate(
        [edge_index[0], loop,
         jnp.zeros((pad,), jnp.int32)]).reshape(NCHUNK, CH)
    dst = jnp.concatenate(
        [edge_index[1], loop,
         jnp.full((pad,), N, jnp.int32)]).reshape(NCHUNK, CH)
    batch_cnt = jnp.concatenate(
        [batch, jnp.full((BP - N,), G, jnp.int32)]).reshape(BP // CH, CH)
    batch_np = jnp.concatenate([batch, jnp.full((NP - N,), G, jnp.int32)])
    W1p = jnp.concatenate([W1, jnp.zeros((1, H), _f32)], axis=0)
    z8 = jnp.zeros((CH, 8), _f32)
    z64 = jnp.zeros((CH, H), _f32)

    srcp, dstp, ecnt = _partition(src, dst)
    srcp2 = srcp.reshape(NC * NREG * (RCAP // CH), CH)
    dstp2 = dstp.reshape(NC * NREG * (RCAP // CH), CH)
    degp, cntp = _deg_counts(dst, batch_cnt)
    dis, t0 = _tc0(degp.reshape(N // 2000, NC * NS, 2000), x)
    a1 = _agg8(z8, srcp2, dstp2, ecnt, t0)
    q1 = _tc_layer(a1, dis, W1p, b1.reshape(1, H))
    a2 = _agg64(z64, srcp2, dstp2, ecnt, q1)
    q2 = _tc_layer(a2, dis, W2, b2.reshape(1, H))
    dis_np = jnp.concatenate([dis.reshape(N), jnp.ones((NP - N,), _f32)])
    p = _agg64_pool(z64, srcp2, dstp2, ecnt, q2, dis_np, batch_np)
    return _tc_final(p.reshape(2, G, H), cntp.reshape(NC * NS, G),
                     W3, b3.reshape(1, H), Wlin, blin.reshape(1, C))


# speedup vs baseline: 2.0683x; 2.0683x over previous
import functools

import jax
import jax.numpy as jnp
from jax import lax
from jax.experimental import pallas as pl
from jax.experimental.pallas import tpu as pltpu
from jax.experimental.pallas import tpu_sc as plsc

N = 50000
F_IN = 7
H = 64
C = 2
G = 512
E = 800000

NC = 2
NS = 16
CH = 128
EP = 851968
NCHUNK = EP // CH
ROWS_PER_SC_TILE = NCHUNK // NS
ROWS_PER_WORKER = NCHUNK // (NC * NS)
BLK = 16
NP = 51200
STRIPE = NP // NS
BP = 65536
GP = 640

_mesh = plsc.VectorSubcoreMesh(core_axis_name="c", subcore_axis_name="s")
_f32 = jnp.float32
_SC_PARAMS = pltpu.CompilerParams(needs_layout_passes=False,
                                  use_tc_tiling_on_sc=False)


EW = E // (NC * NS)
SLW = 2000


@functools.partial(
    pl.kernel,
    out_type=(jax.ShapeDtypeStruct((EP,), jnp.int32),
              jax.ShapeDtypeStruct((EP,), jnp.int32),
              jax.ShapeDtypeStruct((NC * NS * N,), _f32),
              jax.ShapeDtypeStruct((NC * NS * G,), _f32)),
    mesh=_mesh,
    compiler_params=_SC_PARAMS,
    scratch_types=[
        pltpu.VMEM((EW + 8,), jnp.int32),
        pltpu.VMEM((SLW,), jnp.int32),
        pltpu.VMEM((N,), _f32),
        pltpu.VMEM((G,), _f32),
        pltpu.VMEM((BLK, CH), jnp.int32),
    ],
)
def _deg_counts(eidx_hbm, batch_hbm, src_hbm, dst_hbm, degp_hbm, cntp_hbm,
                ebuf, ibuf, deg_v, cnt_v, blk_v):
    c = lax.axis_index("c")
    s = lax.axis_index("s")
    w = s * NC + c
    zero16 = jnp.zeros((16,), _f32)
    one16 = jnp.ones((16,), _f32)
    iota16 = lax.broadcasted_iota(jnp.int32, (16,), 0)

    def _z(i, _):
        deg_v[pl.ds(i * 16, 16)] = zero16
        return 0

    lax.fori_loop(0, N // 16, _z, 0)
    for i in range(G // 16):
        cnt_v[pl.ds(i * 16, 16)] = zero16

    pltpu.sync_copy(eidx_hbm.at[0].at[pl.ds(w * EW, EW)],
                    ebuf.at[pl.ds(0, EW)])
    pltpu.sync_copy(ebuf.at[pl.ds(0, EW)], src_hbm.at[pl.ds(w * EW, EW)])
    pltpu.sync_copy(eidx_hbm.at[1].at[pl.ds(w * EW, EW)],
                    ebuf.at[pl.ds(0, EW)])
    pltpu.sync_copy(ebuf.at[pl.ds(0, EW)], dst_hbm.at[pl.ds(w * EW, EW)])

    def _hist(i, _):
        v = ebuf[pl.ds(i * 16, 16)]
        plsc.addupdate_scatter(deg_v, [v], one16)
        return 0

    lax.fori_loop(0, EW // 16, _hist, 0)
    vtail = ebuf[pl.ds((EW // 16) * 16, 16)]
    plsc.addupdate_scatter(deg_v, [vtail], one16,
                           mask=iota16 < (EW - (EW // 16) * 16))

    @pl.when(w < N // SLW)
    def _():
        base = w * SLW

        def _io(i, _):
            ibuf[pl.ds(i * 16, 16)] = base + i * 16 + iota16
            return 0

        lax.fori_loop(0, SLW // 16, _io, 0)
        pltpu.sync_copy(ibuf, src_hbm.at[pl.ds(E + base, SLW)])
        pltpu.sync_copy(ibuf, dst_hbm.at[pl.ds(E + base, SLW)])

    @pl.when(w == N // SLW)
    def _():
        npad = EP - E - N

        def _ps(i, _):
            ibuf[pl.ds(i * 16, 16)] = jnp.zeros((16,), jnp.int32)
            return 0

        lax.fori_loop(0, npad // 16, _ps, 0)
        pltpu.sync_copy(ibuf.at[pl.ds(0, npad)],
                        src_hbm.at[pl.ds(E + N, npad)])

        def _pd(i, _):
            ibuf[pl.ds(i * 16, 16)] = jnp.full((16,), N, jnp.int32)
            return 0

        lax.fori_loop(0, npad // 16, _pd, 0)
        pltpu.sync_copy(ibuf.at[pl.ds(0, npad)],
                        dst_hbm.at[pl.ds(E + N, npad)])

    nrows = (BP // CH) // (NC * NS)
    pltpu.sync_copy(batch_hbm.at[pl.ds(w * nrows, nrows)], blk_v)
    for r in range(nrows):
        for k in range(CH // 16):
            v = blk_v[r, pl.ds(k * 16, 16)]
            plsc.addupdate_scatter(cnt_v, [v], one16, mask=v < G)

    def _wr(i, _):
        pltpu.sync_copy(deg_v.at[pl.ds(i * 2000, 2000)],
                        degp_hbm.at[pl.ds(i * (NC * NS * 2000) + w * 2000,
                                          2000)])
        return 0

    lax.fori_loop(0, N // 2000, _wr, 0)
    pltpu.sync_copy(cnt_v, cntp_hbm.at[pl.ds(w * G, G)])


@functools.partial(
    pl.kernel,
    out_type=jax.ShapeDtypeStruct((NC * NP, 8), _f32),
    mesh=_mesh,
    compiler_params=_SC_PARAMS,
    scratch_types=[
        pltpu.VMEM_SHARED((NP, 8), _f32),
        pltpu.VMEM((BLK, CH), jnp.int32),
        pltpu.VMEM((BLK, CH), jnp.int32),
        pltpu.VMEM((4, CH, 8), _f32),
        pltpu.SemaphoreType.DMA,
        pltpu.SemaphoreType.DMA,
        pltpu.SemaphoreType.DMA,
        pltpu.SemaphoreType.DMA,
    ],
)
def _agg8(zeros_hbm, src_hbm, dst_hbm, t0_hbm, out_hbm, acc_sh, sb, db,
          rb, gsem0, gsem1, ssem0, ssem1):
    c = lax.axis_index("c")
    s = lax.axis_index("s")
    w = s * NC + c
    gsems = (gsem0, gsem1)
    ssems = (ssem0, ssem1)

    def _zc(i, _):
        pltpu.sync_copy(zeros_hbm, acc_sh.at[pl.ds(s * STRIPE + i * CH, CH)])
        return 0

    lax.fori_loop(0, STRIPE // CH, _zc, 0)
    plsc.subcore_barrier()

    def _blk(b, _):
        row0 = w * ROWS_PER_WORKER + b * BLK
        pltpu.sync_copy(src_hbm.at[pl.ds(row0, BLK)], sb)
        pltpu.sync_copy(dst_hbm.at[pl.ds(row0, BLK)], db)
        gd = [None] * 4
        sd = [None] * BLK
        gd[0] = pltpu.async_copy(t0_hbm.at[sb.at[0]], rb.at[0], gsems[0])
        gd[1] = pltpu.async_copy(t0_hbm.at[sb.at[1]], rb.at[1], gsems[1])
        for j in range(BLK):
            gd[j % 4].wait()
            sd[j] = pltpu.async_copy(rb.at[j % 4], acc_sh.at[db.at[j]],
                                     ssems[j % 2], add=True)
            nj = j + 2
            if nj < BLK:
                if nj >= 4:
                    sd[nj - 4].wait()
                gd[nj % 4] = pltpu.async_copy(t0_hbm.at[sb.at[nj]],
                                              rb.at[nj % 4], gsems[nj % 2])
        for j in range(BLK - 4, BLK):
            sd[j].wait()
        return 0

    lax.fori_loop(0, ROWS_PER_WORKER // BLK, _blk, 0)
    plsc.subcore_barrier()
    pltpu.sync_copy(acc_sh.at[pl.ds(s * STRIPE, STRIPE)],
                    out_hbm.at[pl.ds(c * NP + s * STRIPE, STRIPE)])


def _agg32_body(zeros_hbm, src_hbm, dst_hbm, q_hbm, acc_sh, sb, db, ib, rb,
                gsems, ssems):
    c = lax.axis_index("c")
    s = lax.axis_index("s")

    def _zc(i, _):
        pltpu.sync_copy(zeros_hbm, acc_sh.at[pl.ds(s * STRIPE + i * CH, CH)])
        return 0

    lax.fori_loop(0, STRIPE // CH, _zc, 0)
    plsc.subcore_barrier()

    coff = c * N

    def _blk(b, _):
        row0 = s * ROWS_PER_SC_TILE + b * BLK
        pltpu.sync_copy(src_hbm.at[pl.ds(row0, BLK)], sb)
        pltpu.sync_copy(dst_hbm.at[pl.ds(row0, BLK)], db)

        def _mkidx(j):
            pp = j % 2
            for k in range(CH // 16):
                ib[pp, pl.ds(k * 16, 16)] = sb[j, pl.ds(k * 16, 16)] + coff

        gd = [None] * 4
        sd = [None] * BLK
        _mkidx(0)
        gd[0] = pltpu.async_copy(q_hbm.at[ib.at[0]], rb.at[0], gsems[0])
        _mkidx(1)
        gd[1] = pltpu.async_copy(q_hbm.at[ib.at[1]], rb.at[1], gsems[1])
        for j in range(BLK):
            gd[j % 4].wait()
            sd[j] = pltpu.async_copy(rb.at[j % 4], acc_sh.at[db.at[j]],
                                     ssems[j % 2], add=True)
            nj = j + 2
            if nj < BLK:
                if nj >= 4:
                    sd[nj - 4].wait()
                _mkidx(nj)
                gd[nj % 4] = pltpu.async_copy(q_hbm.at[ib.at[nj % 2]],
                                              rb.at[nj % 4], gsems[nj % 2])
        for j in range(BLK - 4, BLK):
            sd[j].wait()
        return 0

    lax.fori_loop(0, ROWS_PER_SC_TILE // BLK, _blk, 0)


_AGG32_SCRATCH = [
    pltpu.VMEM_SHARED((NP, 32), _f32),
    pltpu.VMEM((BLK, CH), jnp.int32),
    pltpu.VMEM((BLK, CH), jnp.int32),
    pltpu.VMEM((2, CH), jnp.int32),
    pltpu.VMEM((4, CH, 32), _f32),
    pltpu.SemaphoreType.DMA,
    pltpu.SemaphoreType.DMA,
    pltpu.SemaphoreType.DMA,
    pltpu.SemaphoreType.DMA,
]


@functools.partial(
    pl.kernel,
    out_type=jax.ShapeDtypeStruct((NC * NP, 32), _f32),
    mesh=_mesh,
    compiler_params=_SC_PARAMS,
    scratch_types=_AGG32_SCRATCH,
)
def _agg32(zeros_hbm, src_hbm, dst_hbm, q_hbm, out_hbm, acc_sh, sb, db,
           ib, rb, gsem0, gsem1, ssem0, ssem1):
    c = lax.axis_index("c")
    s = lax.axis_index("s")
    _agg32_body(zeros_hbm, src_hbm, dst_hbm, q_hbm, acc_sh, sb, db, ib, rb,
                (gsem0, gsem1), (ssem0, ssem1))
    plsc.subcore_barrier()
    pltpu.sync_copy(acc_sh.at[pl.ds(s * STRIPE, STRIPE)],
                    out_hbm.at[pl.ds(c * NP + s * STRIPE, STRIPE)])


@functools.partial(
    pl.kernel,
    out_type=jax.ShapeDtypeStruct((NC * G, 32), _f32),
    mesh=_mesh,
    compiler_params=_SC_PARAMS,
    scratch_types=_AGG32_SCRATCH + [
        pltpu.VMEM_SHARED((GP, 32), _f32),
        pltpu.VMEM((CH,), _f32),
        pltpu.VMEM((CH,), jnp.int32),
        pltpu.VMEM((CH, 32), _f32),
    ],
)
def _agg32_pool(zeros_hbm, src_hbm, dst_hbm, q_hbm, dis_hbm, batch_hbm,
                out_hbm, acc_sh, sb, db, ib, rb, gsem0, gsem1, ssem0,
                ssem1, pool_sh, dbuf, bidx, pz):
    c = lax.axis_index("c")
    s = lax.axis_index("s")
    gstripe = GP // NS
    pltpu.sync_copy(zeros_hbm.at[pl.ds(0, gstripe)],
                    pool_sh.at[pl.ds(s * gstripe, gstripe)])
    _agg32_body(zeros_hbm, src_hbm, dst_hbm, q_hbm, acc_sh, sb, db, ib, rb,
                (gsem0, gsem1), (ssem0, ssem1))
    plsc.subcore_barrier()

    def _pchunk(t, _):
        node0 = s * STRIPE + t * CH
        pltpu.sync_copy(acc_sh.at[pl.ds(node0, CH)], pz)
        pltpu.sync_copy(dis_hbm.at[pl.ds(node0, CH)], dbuf)
        pltpu.sync_copy(batch_hbm.at[pl.ds(node0, CH)], bidx)

        def _srow(j, _):
            idx16 = lax.broadcast(j, (16,))
            dsp = plsc.load_gather(dbuf, [idx16])
            pz[j, pl.ds(0, 16)] = pz[j, pl.ds(0, 16)] * dsp
            pz[j, pl.ds(16, 16)] = pz[j, pl.ds(16, 16)] * dsp
            return 0

        lax.fori_loop(0, CH, _srow, 0)
        pltpu.sync_copy(pz, pool_sh.at[bidx], add=True)
        return 0

    lax.fori_loop(0, STRIPE // CH, _pchunk, 0)
    plsc.subcore_barrier()
    gd = G // NS
    pltpu.sync_copy(pool_sh.at[pl.ds(s * gd, gd)],
                    out_hbm.at[pl.ds(c * G + s * gd, gd)])


_NB = 2000
_NBLK = N // _NB


def _tc0_body(degp_ref, x_ref, dis_ref, t0_ref):
    deg = jnp.sum(degp_ref[0], axis=0) + 1.0
    dis = lax.rsqrt(deg)
    dis_ref[...] = dis[:, None]
    t0 = x_ref[...] * dis[:, None]
    t0_ref[...] = jnp.concatenate([t0, jnp.zeros((_NB, 1), _f32)], axis=1)


def _tc0(degp, x):
    return pl.pallas_call(
        _tc0_body,
        grid=(_NBLK,),
        in_specs=[
            pl.BlockSpec((1, NC * NS, _NB), lambda i: (i, 0, 0)),
            pl.BlockSpec((_NB, F_IN), lambda i: (i, 0)),
        ],
        out_specs=[
            pl.BlockSpec((_NB, 1), lambda i: (i, 0)),
            pl.BlockSpec((_NB, 8), lambda i: (i, 0)),
        ],
        out_shape=[
            jax.ShapeDtypeStruct((N, 1), _f32),
            jax.ShapeDtypeStruct((N, 8), _f32),
        ],
    )(degp, x)


def _tc_layer_body(split_in, ap_ref, dis_ref, w_ref, b_ref, q_ref):
    if split_in:
        a = jnp.concatenate([ap_ref[0], ap_ref[1]], axis=1)
    else:
        a = ap_ref[0] + ap_ref[1]
    dis = dis_ref[...]
    z = a * dis
    h = jnp.dot(z, w_ref[...], preferred_element_type=_f32)
    h = jnp.maximum(h + b_ref[...], 0.0)
    q = h * dis
    q_ref[0] = q[:, :32]
    q_ref[1] = q[:, 32:]


def _tc_layer(ap, dis, w, b, split_in):
    kin = ap.shape[-1]
    return pl.pallas_call(
        functools.partial(_tc_layer_body, split_in),
        grid=(_NBLK,),
        in_specs=[
            pl.BlockSpec((2, _NB, kin), lambda i: (0, i, 0)),
            pl.BlockSpec((_NB, 1), lambda i: (i, 0)),
            pl.BlockSpec(w.shape, lambda i: (0, 0)),
            pl.BlockSpec((1, H), lambda i: (0, 0)),
        ],
        out_specs=pl.BlockSpec((2, _NB, 32), lambda i: (0, i, 0)),
        out_shape=jax.ShapeDtypeStruct((2, N, 32), _f32),
    )(ap, dis, w, b)


def _tc_final_body(p_ref, cntp_ref, w3_ref, b3_ref, wl_ref, bl_ref, out_ref):
    counts = jnp.sum(cntp_ref[...], axis=0)
    p64 = jnp.concatenate([p_ref[0], p_ref[1]], axis=1)
    hs = jnp.dot(p64, w3_ref[...], preferred_element_type=_f32)
    hs = hs + counts[:, None] * b3_ref[...]
    pooled = hs / jnp.maximum(counts, 1.0)[:, None]
    out = jnp.dot(pooled, wl_ref[...], preferred_element_type=_f32)
    out_ref[...] = out + bl_ref[...]


def _tc_final(p, cntp, w3, b3, wl, bl):
    return pl.pallas_call(
        _tc_final_body,
        out_shape=jax.ShapeDtypeStruct((G, C), _f32),
    )(p, cntp, w3, b3, wl, bl)


def kernel(x, edge_index, batch, W1, b1, W2, b2, W3, b3, Wlin, blin):
    batch_cnt = jnp.concatenate(
        [batch, jnp.full((BP - N,), G, jnp.int32)]).reshape(BP // CH, CH)
    batch_np = jnp.concatenate([batch, jnp.full((NP - N,), G, jnp.int32)])
    W1p = jnp.concatenate([W1, jnp.zeros((1, H), _f32)], axis=0)
    z8 = jnp.zeros((CH, 8), _f32)
    z32 = jnp.zeros((CH, 32), _f32)

    srcf, dstf, degp, cntp = _deg_counts(edge_index, batch_cnt)
    src = srcf.reshape(NCHUNK, CH)
    dst = dstf.reshape(NCHUNK, CH)
    dis, t0 = _tc0(degp.reshape(N // 2000, NC * NS, 2000), x)
    a1p = _agg8(z8, src, dst, t0)
    q1 = _tc_layer(a1p.reshape(2, NP, 8), dis, W1p, b1.reshape(1, H), False)
    a2 = _agg32(z32, src, dst, q1.reshape(NC * N, 32))
    q2 = _tc_layer(a2.reshape(2, NP, 32), dis, W2, b2.reshape(1, H), True)
    dis_np = jnp.concatenate([dis.reshape(N), jnp.ones((NP - N,), _f32)])
    p = _agg32_pool(z32, src, dst, q2.reshape(NC * N, 32), dis_np, batch_np)
    return _tc_final(p.reshape(2, G, 32), cntp.reshape(NC * NS, G),
                     W3, b3.reshape(1, H), Wlin, blin.reshape(1, C))
